# baseline (device time: 172850 ns/iter reference)
import jax
import jax.numpy as jnp
from jax import lax
from jax.experimental import pallas as pl
from jax.experimental.pallas import tpu as pltpu

N_DEV = 8
DEPTH = 3
SUB = 2


def kernel(x, w_mat):
    m_per, k = x.shape
    _, n_per = w_mat.shape
    half = m_per // 2
    sub = half // SUB

    def body(x_ref, w_ref, out_ref, buf_a, buf_b, w_bf,
             x_stg, w_stg, in_sems,
             send_a, recv_a, send_b, recv_b, credit_a, credit_b):
        my = lax.axis_index("i")
        left = (my + N_DEV - 1) % N_DEV
        right = (my + 1) % N_DEV

        barrier_sem = pltpu.get_barrier_semaphore()
        for nbr in (left, right):
            pl.semaphore_signal(
                barrier_sem, inc=1,
                device_id=(nbr,), device_id_type=pl.DeviceIdType.MESH,
            )
        pl.semaphore_wait(barrier_sem, 2)

        def rdma(buf, send_sems, recv_sems, j, h, dev):
            s = h % DEPTH
            r = (h + 1) % DEPTH
            return pltpu.make_async_remote_copy(
                src_ref=buf.at[s, pl.ds(j * sub, sub), :],
                dst_ref=buf.at[r, pl.ds(j * sub, sub), :],
                send_sem=send_sems.at[j, h],
                recv_sem=recv_sems.at[j, h],
                device_id=(dev,),
                device_id_type=pl.DeviceIdType.MESH,
            )

        def rdma_a(j, h):
            return rdma(buf_a, send_a, recv_a, j, h, right)

        def rdma_b(j, h):
            return rdma(buf_b, send_b, recv_b, j, h, left)

        def gemm_store(h):
            s = h % DEPTH
            origin_a = (my - h) % N_DEV
            origin_b = (my + h) % N_DEV
            ya = jnp.dot(buf_a[s], w_bf[...],
                         preferred_element_type=jnp.float32)
            out_ref[pl.ds(origin_a * m_per, half), :] = jnp.maximum(ya, 0.0)
            yb = jnp.dot(buf_b[s], w_bf[...],
                         preferred_element_type=jnp.float32)
            out_ref[pl.ds(origin_b * m_per + half, half), :] = (
                jnp.maximum(yb, 0.0))

        piece_rows = (0, half, sub, half + sub)

        def x_dma(p):
            return pltpu.make_async_copy(
                x_ref.at[pl.ds(piece_rows[p], sub), :],
                x_stg.at[p],
                in_sems.at[p],
            )

        w_dma = pltpu.make_async_copy(w_ref, w_stg, in_sems.at[4])

        x_dma(0).start()
        x_dma(1).start()
        x_dma(0).wait()
        buf_a[0, :sub, :] = x_stg[0].astype(jnp.bfloat16)
        rdma_a(0, 0).start()
        x_dma(2).start()
        x_dma(3).start()
        w_dma.start()
        x_dma(1).wait()
        buf_b[0, :sub, :] = x_stg[1].astype(jnp.bfloat16)
        rdma_b(0, 0).start()
        x_dma(2).wait()
        buf_a[0, sub:, :] = x_stg[2].astype(jnp.bfloat16)
        rdma_a(1, 0).start()
        x_dma(3).wait()
        buf_b[0, sub:, :] = x_stg[3].astype(jnp.bfloat16)
        rdma_b(1, 0).start()
        w_dma.wait()
        w_bf[...] = w_stg[...].astype(jnp.bfloat16)
        gemm_store(0)

        for h in range(1, N_DEV - 1):
            if h >= 2:
                pl.semaphore_wait(credit_a, 1)
                pl.semaphore_wait(credit_b, 1)
            rdma_a(0, h - 1).wait_recv()
            rdma_a(0, h).start()
            rdma_b(0, h - 1).wait_recv()
            rdma_b(0, h).start()
            rdma_a(1, h - 1).wait_recv()
            rdma_a(1, h).start()
            rdma_b(1, h - 1).wait_recv()
            rdma_b(1, h).start()
            for j in range(SUB):
                rdma_a(j, h - 1).wait_send()
                rdma_b(j, h - 1).wait_send()
            if h <= 5:
                pl.semaphore_signal(
                    credit_a, inc=1,
                    device_id=(left,), device_id_type=pl.DeviceIdType.MESH,
                )
                pl.semaphore_signal(
                    credit_b, inc=1,
                    device_id=(right,), device_id_type=pl.DeviceIdType.MESH,
                )
            gemm_store(h)

        def gemm_store_sub(buf, origin_row, j):
            s = (N_DEV - 1) % DEPTH
            y = jnp.dot(buf[s, pl.ds(j * sub, sub), :], w_bf[...],
                        preferred_element_type=jnp.float32)
            out_ref[pl.ds(origin_row + j * sub, sub), :] = (
                jnp.maximum(y, 0.0))

        origin_a = (my - (N_DEV - 1)) % N_DEV
        origin_b = (my + (N_DEV - 1)) % N_DEV
        for j in range(SUB):
            rdma_a(j, N_DEV - 2).wait_recv()
            rdma_b(j, N_DEV - 2).wait_recv()
            gemm_store_sub(buf_a, origin_a * m_per, j)
            gemm_store_sub(buf_b, origin_b * m_per + half, j)
        for j in range(SUB):
            rdma_a(j, N_DEV - 2).wait_send()
            rdma_b(j, N_DEV - 2).wait_send()

    return pl.pallas_call(
        body,
        out_shape=jax.ShapeDtypeStruct((N_DEV * m_per, n_per), jnp.float32),
        in_specs=[
            pl.BlockSpec(memory_space=pltpu.MemorySpace.HBM),
            pl.BlockSpec(memory_space=pltpu.MemorySpace.HBM),
        ],
        out_specs=pl.BlockSpec(memory_space=pltpu.VMEM),
        scratch_shapes=[
            pltpu.VMEM((DEPTH, half, k), jnp.bfloat16),
            pltpu.VMEM((DEPTH, half, k), jnp.bfloat16),
            pltpu.VMEM((k, n_per), jnp.bfloat16),
            pltpu.VMEM((4, sub, k), jnp.float32),
            pltpu.VMEM((k, n_per), jnp.float32),
            pltpu.SemaphoreType.DMA((5,)),
            pltpu.SemaphoreType.DMA((SUB, N_DEV - 1)),
            pltpu.SemaphoreType.DMA((SUB, N_DEV - 1)),
            pltpu.SemaphoreType.DMA((SUB, N_DEV - 1)),
            pltpu.SemaphoreType.DMA((SUB, N_DEV - 1)),
            pltpu.SemaphoreType.REGULAR,
            pltpu.SemaphoreType.REGULAR,
        ],
        compiler_params=pltpu.CompilerParams(collective_id=0),
    )(x, w_mat)
